# baseline (speedup 1.0000x reference)
import functools

import jax
import jax.numpy as jnp
from jax import lax
from jax.experimental import pallas as pl
from jax.experimental.pallas import tpu as pltpu
from jax.experimental.pallas import tpu_sc as plsc

N = 100000
E = 1600000
D = 32
NC = 2
NS = 16

HALF = N // 2
SUM_ACC_R = HALF + 48
EPT = E // NS
CH = 80

BLK = N // (NC * NS)
MACC_R = BLK + 16
MAFL = MACC_R * D
MOUT = (BLK + 3) * D
MCH = 800
MP = 128
MCAP = MP + MCH + 16

_ROWBLK = 2000
_GRID = N // _ROWBLK


def _mesh():
  return plsc.VectorSubcoreMesh(core_axis_name="c", subcore_axis_name="s",
                                num_cores=NC, num_subcores=NS)


@functools.partial(
    pl.kernel,
    out_type=jax.ShapeDtypeStruct((NC, SUM_ACC_R, D), jnp.float32),
    mesh=_mesh(),
    scratch_types=[
        pltpu.VMEM_SHARED((SUM_ACC_R, D), jnp.float32),
        pltpu.VMEM((CH,), jnp.int32),
        pltpu.VMEM((CH,), jnp.int32),
        pltpu.VMEM((CH,), jnp.int32),
        pltpu.VMEM((CH, 128), jnp.float32),
        pltpu.VMEM((CH, D), jnp.float32),
        pltpu.VMEM((16,), jnp.int32),
        pltpu.VMEM((16, D), jnp.float32),
        pltpu.SemaphoreType.DMA,
    ],
    compiler_params=pltpu.CompilerParams(needs_layout_passes=False),
)
def _sc_segsum(xs_hbm, src_hbm, dst_hbm, out_hbm,
               acc_sh, src_v, dst_v, idx_v, rows_v, rows32_v, idx8_v,
               rows8_v, sem):
  c = lax.axis_index("c")
  s = lax.axis_index("s")
  base = c * HALF
  zrows = SUM_ACC_R // NS
  lanes = lax.iota(jnp.int32, 16)
  zv = jnp.zeros((16,), jnp.float32)

  def zfill(i, _):
    rows32_v[i, pl.ds(0, 16)] = zv
    rows32_v[i, pl.ds(16, 16)] = zv
    return 0

  lax.fori_loop(0, CH, zfill, 0)

  def mkidx(row0, clamp):
    def bidx(g, _):
      v = row0 + g * 16 + lanes
      if clamp:
        v = jnp.minimum(v, SUM_ACC_R - 1)
      idx_v[pl.ds(g * 16, 16)] = v
      return 0

    lax.fori_loop(0, CH // 16, bidx, 0)

  def zinit(k, _):
    mkidx(s * zrows + k * CH, True)
    pltpu.sync_copy(rows32_v, acc_sh.at[idx_v])
    return 0

  lax.fori_loop(0, (zrows + CH - 1) // CH, zinit, 0)
  plsc.subcore_barrier()

  def chunk(i, _):
    off = s * EPT + i * CH
    pltpu.sync_copy(src_hbm.at[pl.ds(off, CH)], src_v)
    pltpu.sync_copy(dst_hbm.at[pl.ds(off, CH)], dst_v)

    def vb(g, _):
      d = dst_v[pl.ds(g * 16, 16)]
      dp = d - base
      owned = plsc.bitcast(dp, jnp.uint32) < jnp.uint32(HALF)
      idx_v[pl.ds(g * 16, 16)] = jnp.where(owned, dp, HALF + lanes)
      return 0

    lax.fori_loop(0, CH // 16, vb, 0)
    pltpu.async_copy(xs_hbm.at[src_v], rows_v, sem).wait()

    def repack(e, _):
      rows32_v[e, pl.ds(0, 16)] = rows_v[e, pl.ds(0, 16)]
      rows32_v[e, pl.ds(16, 16)] = rows_v[e, pl.ds(16, 16)]
      return 0

    lax.fori_loop(0, CH, repack, 0)
    pltpu.sync_copy(rows32_v, acc_sh.at[idx_v], add=True)
    return 0

  lax.fori_loop(0, EPT // CH, chunk, 0)
  plsc.subcore_barrier()

  def wb(k, _):
    mkidx(s * zrows + k * CH, False)
    pltpu.async_copy(acc_sh.at[idx_v], rows32_v, sem).wait()
    pltpu.sync_copy(rows32_v, out_hbm.at[c, pl.ds(s * zrows + k * CH, CH)])
    return 0

  lax.fori_loop(0, zrows // CH, wb, 0)
  toff = s * zrows + (zrows // CH) * CH
  idx8_v[...] = jnp.minimum(toff + lanes, SUM_ACC_R - 1)
  pltpu.async_copy(acc_sh.at[idx8_v], rows8_v, sem).wait()
  pltpu.sync_copy(rows8_v.at[pl.ds(0, zrows % CH)],
                  out_hbm.at[c, pl.ds(toff, zrows % CH)])


@functools.partial(
    pl.kernel,
    out_type=(jax.ShapeDtypeStruct((NC, SUM_ACC_R, 16), jnp.float32),
              jax.ShapeDtypeStruct((NC, SUM_ACC_R, 16), jnp.float32)),
    mesh=_mesh(),
    scratch_types=[
        pltpu.VMEM_SHARED((SUM_ACC_R, 16), jnp.float32),
        pltpu.VMEM_SHARED((SUM_ACC_R, 16), jnp.float32),
        pltpu.VMEM((CH,), jnp.int32),
        pltpu.VMEM((CH,), jnp.int32),
        pltpu.VMEM((CH, 16), jnp.float32),
        pltpu.VMEM((CH, 16), jnp.float32),
        pltpu.VMEM((16,), jnp.int32),
        pltpu.VMEM((16, 16), jnp.float32),
        pltpu.SemaphoreType.DMA,
    ],
    compiler_params=pltpu.CompilerParams(needs_layout_passes=False),
)
def _sc_counts(dst1_hbm, dst3_hbm, out1_hbm, out3_hbm,
               acc1_sh, acc3_sh, dst_v, idx_v, ones_v, zb_v, idx8_v,
               rows8_v, sem):
  c = lax.axis_index("c")
  s = lax.axis_index("s")
  base = c * HALF
  zrows = SUM_ACC_R // NS
  lanes = lax.iota(jnp.int32, 16)

  one = jnp.full((16,), 1.0, jnp.float32)
  zv = jnp.zeros((16,), jnp.float32)

  def fill(i, _):
    ones_v[i, :] = one
    zb_v[i, :] = zv
    return 0

  lax.fori_loop(0, CH, fill, 0)

  def mkidx(row0, clamp):
    def bidx(g, _):
      v = row0 + g * 16 + lanes
      if clamp:
        v = jnp.minimum(v, SUM_ACC_R - 1)
      idx_v[pl.ds(g * 16, 16)] = v
      return 0

    lax.fori_loop(0, CH // 16, bidx, 0)

  for acc_sh in (acc1_sh, acc3_sh):

    def zinit(k, _):
      mkidx(s * zrows + k * CH, True)
      pltpu.sync_copy(zb_v, acc_sh.at[idx_v])
      return 0

    lax.fori_loop(0, (zrows + CH - 1) // CH, zinit, 0)

  plsc.subcore_barrier()

  for dst_hbm, acc_sh in ((dst1_hbm, acc1_sh), (dst3_hbm, acc3_sh)):

    def chunk(i, _):
      off = s * EPT + i * CH
      pltpu.sync_copy(dst_hbm.at[pl.ds(off, CH)], dst_v)

      def vb(g, _):
        d = dst_v[pl.ds(g * 16, 16)]
        dp = d - base
        owned = plsc.bitcast(dp, jnp.uint32) < jnp.uint32(HALF)
        idx_v[pl.ds(g * 16, 16)] = jnp.where(owned, dp, HALF + lanes)
        return 0

      lax.fori_loop(0, CH // 16, vb, 0)
      pltpu.sync_copy(ones_v, acc_sh.at[idx_v], add=True)
      return 0

    lax.fori_loop(0, EPT // CH, chunk, 0)

  plsc.subcore_barrier()

  for acc_sh, o_hbm in ((acc1_sh, out1_hbm), (acc3_sh, out3_hbm)):

    def wb(k, _):
      mkidx(s * zrows + k * CH, False)
      pltpu.async_copy(acc_sh.at[idx_v], zb_v, sem).wait()
      pltpu.sync_copy(zb_v, o_hbm.at[c, pl.ds(s * zrows + k * CH, CH)])
      return 0

    lax.fori_loop(0, zrows // CH, wb, 0)
    toff = s * zrows + (zrows // CH) * CH
    idx8_v[...] = jnp.minimum(toff + lanes, SUM_ACC_R - 1)
    pltpu.async_copy(acc_sh.at[idx8_v], rows8_v, sem).wait()
    pltpu.sync_copy(rows8_v.at[pl.ds(0, zrows % CH)],
                    o_hbm.at[c, pl.ds(toff, zrows % CH)])


@functools.partial(
    pl.kernel,
    out_type=jax.ShapeDtypeStruct((NC * NS * MOUT,), jnp.float32),
    mesh=_mesh(),
    scratch_types=[
        pltpu.VMEM((MAFL,), jnp.float32),
        pltpu.VMEM((MCH,), jnp.int32),
        pltpu.VMEM((MCH,), jnp.int32),
        pltpu.VMEM((MCAP,), jnp.int32),
        pltpu.VMEM((MCAP,), jnp.int32),
        pltpu.VMEM((MP, 128), jnp.float32),
        pltpu.SemaphoreType.DMA,
    ],
    compiler_params=pltpu.CompilerParams(needs_layout_passes=False),
)
def _sc_segmax(y_hbm, src_hbm, dst_hbm, ninf_hbm, out_hbm,
               acc_v, src_v, dst_v, csrc_v, cdst_v, rows_v, sem):
  c = lax.axis_index("c")
  s = lax.axis_index("s")
  wid = s * NC + c
  base = wid * BLK

  pltpu.sync_copy(ninf_hbm, acc_v)

  lanes = lax.iota(jnp.int32, 16)

  def initc(g, _):
    cdst_v[pl.ds(g * 16, 16)] = BLK + lanes
    csrc_v[pl.ds(g * 16, 16)] = lanes * 512 + g
    return 0

  lax.fori_loop(0, MCAP // 16, initc, 0)

  def fire():
    pltpu.async_copy(y_hbm.at[csrc_v.at[pl.ds(0, MP)]], rows_v, sem).wait()

    def rmw(e, _):
      dp = cdst_v[pl.ds(e, 16)][0]
      a0 = acc_v[pl.ds(dp * D, 16)]
      r0 = rows_v[e, pl.ds(0, 16)]
      acc_v[pl.ds(dp * D, 16)] = jnp.maximum(a0, r0)
      a1 = acc_v[pl.ds(dp * D + 16, 16)]
      r1 = rows_v[e, pl.ds(16, 16)]
      acc_v[pl.ds(dp * D + 16, 16)] = jnp.maximum(a1, r1)
      return 0

    lax.fori_loop(0, MP, rmw, 0)

    def shift(g, _):
      cdst_v[pl.ds(g * 16, 16)] = cdst_v[pl.ds(MP + g * 16, 16)]
      csrc_v[pl.ds(g * 16, 16)] = csrc_v[pl.ds(MP + g * 16, 16)]
      return 0

    lax.fori_loop(0, (MCAP - MP) // 16, shift, 0)

  def fire_n(n):
    lax.fori_loop(0, n, lambda i, _: (fire(), 0)[1], 0)

  def chunk(i, cur):
    off = i * MCH
    pltpu.sync_copy(src_hbm.at[pl.ds(off, MCH)], src_v)
    pltpu.sync_copy(dst_hbm.at[pl.ds(off, MCH)], dst_v)

    def vb(g, cur):
      d = dst_v[pl.ds(g * 16, 16)]
      sv = src_v[pl.ds(g * 16, 16)]
      dp = d - base
      owned = plsc.bitcast(dp, jnp.uint32) < jnp.uint32(BLK)
      cs = plsc.cumsum(jnp.where(owned, jnp.int32(1), jnp.int32(0)))
      pos = jnp.where(owned, cur + cs - 1, jnp.int32(MCAP - 16) + lanes)
      plsc.store_scatter(cdst_v, [pos], jnp.where(owned, dp, jnp.int32(BLK)))
      plsc.store_scatter(csrc_v, [pos], jnp.where(owned, sv, lanes * 97 + 8))
      return cur + cs[15]

    cur = lax.fori_loop(0, MCH // 16, vb, cur)
    nf = cur // MP
    fire_n(nf)
    return cur - nf * MP

  cur = lax.fori_loop(0, E // MCH, chunk, jnp.int32(0))
  fire()

  pltpu.sync_copy(acc_v.at[pl.ds(0, MOUT)], out_hbm.at[pl.ds(wid * MOUT, MOUT)])


def _prep_body(x_ref, sid_ref, w4_ref, w5_ref, xs_ref, xsp_ref, y4_ref,
               y5_ref):
  xs = x_ref[...] * (1.0 + sid_ref[...].astype(jnp.float32))
  xs_ref[...] = xs
  pad = jnp.zeros((xs.shape[0], 128 - D), jnp.float32)
  xsp_ref[...] = jnp.concatenate([xs, pad], axis=1)
  y4 = jnp.dot(xs, w4_ref[...], preferred_element_type=jnp.float32)
  y5 = jnp.dot(xs, w5_ref[...], preferred_element_type=jnp.float32)
  y4_ref[...] = jnp.concatenate([y4, pad], axis=1)
  y5_ref[...] = jnp.concatenate([y5, pad], axis=1)


def _prep(x, sid, w4, w5):
  blk = _ROWBLK
  return pl.pallas_call(
      _prep_body,
      grid=(_GRID,),
      in_specs=[
          pl.BlockSpec((blk, D), lambda i: (i, 0)),
          pl.BlockSpec((blk, 1), lambda i: (i, 0)),
          pl.BlockSpec((D, D), lambda i: (0, 0)),
          pl.BlockSpec((D, D), lambda i: (0, 0)),
      ],
      out_specs=[
          pl.BlockSpec((blk, D), lambda i: (i, 0)),
          pl.BlockSpec((blk, 128), lambda i: (i, 0)),
          pl.BlockSpec((blk, 128), lambda i: (i, 0)),
          pl.BlockSpec((blk, 128), lambda i: (i, 0)),
      ],
      out_shape=[jax.ShapeDtypeStruct((N, D), jnp.float32),
                 jax.ShapeDtypeStruct((N, 128), jnp.float32),
                 jax.ShapeDtypeStruct((N, 128), jnp.float32),
                 jax.ShapeDtypeStruct((N, 128), jnp.float32)],
  )(x, sid, w4, w5)


def _finish_body(xs_ref, s1_ref, s2_ref, s3_ref, c1_ref, c3_ref, m4_ref,
                 m5_ref, w1_ref, w2_ref, w3_ref, lhw_ref, lhb_ref, dw1_ref,
                 db1_ref, dw2_ref, db2_ref, out_ref):
  f32 = jnp.float32
  c1 = jnp.clip(c1_ref[...][:, 0:1], 1.0, None)
  c3 = jnp.clip(c3_ref[...][:, 0:1], 1.0, None)
  m4 = m4_ref[...]
  m4 = jnp.where(jnp.isfinite(m4), m4, 0.0)
  m5 = m5_ref[...]
  m5 = jnp.where(jnp.isfinite(m5), m5, 0.0)
  h = (xs_ref[...]
       + jnp.dot(s1_ref[...], w1_ref[...], preferred_element_type=f32) / c1
       + jnp.dot(s2_ref[...], w2_ref[...], preferred_element_type=f32)
       + jnp.dot(s3_ref[...], w3_ref[...], preferred_element_type=f32) / c3
       + m4 + m5)
  h = jnp.maximum(h, 0.0)
  feat = jnp.dot(h, lhw_ref[...], preferred_element_type=f32) + lhb_ref[...]
  z = jnp.maximum(
      jnp.dot(feat, dw1_ref[...], preferred_element_type=f32) + db1_ref[...],
      0.0)
  logit = jnp.dot(z, dw2_ref[...], preferred_element_type=f32) + db2_ref[...]
  out_ref[...] = jax.nn.sigmoid(logit)


def _finish(xs, s1, s2, s3, c1, c3, m4, m5, w1, w2, w3, lhw, lhb, dw1, db1,
            dw2, db2):
  blk = _ROWBLK
  row = lambda r, cdim: pl.BlockSpec((blk, cdim), lambda i: (i, 0))
  full = lambda a: pl.BlockSpec(a.shape, lambda i: (0,) * a.ndim)
  return pl.pallas_call(
      _finish_body,
      grid=(_GRID,),
      in_specs=[
          row(xs, D), row(s1, D), row(s2, D), row(s3, D),
          row(c1, 16), row(c3, 16), row(m4, D), row(m5, D),
          full(w1), full(w2), full(w3), full(lhw), full(lhb),
          full(dw1), full(db1), full(dw2), full(db2),
      ],
      out_specs=pl.BlockSpec((blk, 1), lambda i: (i, 0)),
      out_shape=jax.ShapeDtypeStruct((N, 1), jnp.float32),
  )(xs, s1, s2, s3, c1, c3, m4, m5, w1, w2, w3, lhw, lhb, dw1, db1, dw2, db2)


def kernel(x_stroke, edge_intersects, edge_temp_previous, edge_represented_by,
           edge_brepcoplanar, edge_strokecoplanar, sketch_strokes_id,
           W_intersects, W_temp_previous, W_represented_by, W_brepcoplanar,
           W_strokecoplanar, local_head_w, local_head_b, dec_w1, dec_b1,
           dec_w2, dec_b2):
  xs, xsp, y4, y5 = _prep(x_stroke, sketch_strokes_id, W_brepcoplanar,
                          W_strokecoplanar)

  ninf = jnp.full((MAFL,), -jnp.inf, jnp.float32)

  s1 = _sc_segsum(xsp, edge_intersects[0], edge_intersects[1])
  s2 = _sc_segsum(xsp, edge_temp_previous[0], edge_temp_previous[1])
  s3 = _sc_segsum(xsp, edge_represented_by[0], edge_represented_by[1])
  c1, c3 = _sc_counts(edge_intersects[1], edge_represented_by[1])
  m4 = _sc_segmax(y4, edge_brepcoplanar[0], edge_brepcoplanar[1], ninf)
  m5 = _sc_segmax(y5, edge_strokecoplanar[0], edge_strokecoplanar[1], ninf)

  s1, s2, s3 = (t[:, :HALF].reshape(N, D) for t in (s1, s2, s3))
  c1, c3 = (t[:, :HALF].reshape(N, 16) for t in (c1, c3))
  m4, m5 = (t.reshape(NC * NS, BLK + 3, D)[:, :BLK].reshape(N, D)
            for t in (m4, m5))

  return _finish(xs, s1, s2, s3, c1, c3, m4, m5, W_intersects,
                 W_temp_previous, W_represented_by, local_head_w,
                 local_head_b.reshape(1, -1), dec_w1, dec_b1.reshape(1, -1),
                 dec_w2, dec_b2.reshape(1, -1))


# speedup vs baseline: 1.0847x; 1.0847x over previous
import functools

import jax
import jax.numpy as jnp
from jax import lax
from jax.experimental import pallas as pl
from jax.experimental.pallas import tpu as pltpu
from jax.experimental.pallas import tpu_sc as plsc

N = 100000
E = 1600000
D = 32
NC = 2
NS = 16

HALF = N // 2
SUM_ACC_R = HALF + 48
EPT = E // NS
CH = 80

BLK = N // (NC * NS)
MACC_R = BLK + 16
MAFL = MACC_R * D
MOUT = (BLK + 3) * D
MCH = 1600
MP = 128
MCAP = MP + MCH + 16

_ROWBLK = 2000
_GRID = N // _ROWBLK


def _mesh():
  return plsc.VectorSubcoreMesh(core_axis_name="c", subcore_axis_name="s",
                                num_cores=NC, num_subcores=NS)


@functools.partial(
    pl.kernel,
    out_type=jax.ShapeDtypeStruct((NC, SUM_ACC_R, D), jnp.float32),
    mesh=_mesh(),
    scratch_types=[
        pltpu.VMEM_SHARED((SUM_ACC_R, D), jnp.float32),
        pltpu.VMEM((CH,), jnp.int32),
        pltpu.VMEM((CH,), jnp.int32),
        pltpu.VMEM((CH,), jnp.int32),
        pltpu.VMEM((CH, 128), jnp.float32),
        pltpu.VMEM((CH, D), jnp.float32),
        pltpu.VMEM((16,), jnp.int32),
        pltpu.VMEM((16, D), jnp.float32),
        pltpu.SemaphoreType.DMA,
    ],
    compiler_params=pltpu.CompilerParams(needs_layout_passes=False),
)
def _sc_segsum(xs_hbm, src_hbm, dst_hbm, out_hbm,
               acc_sh, src_v, dst_v, idx_v, rows_v, rows32_v, idx8_v,
               rows8_v, sem):
  c = lax.axis_index("c")
  s = lax.axis_index("s")
  base = c * HALF
  zrows = SUM_ACC_R // NS
  lanes = lax.iota(jnp.int32, 16)
  zv = jnp.zeros((16,), jnp.float32)

  def zfill(i, _):
    rows32_v[i, pl.ds(0, 16)] = zv
    rows32_v[i, pl.ds(16, 16)] = zv
    return 0

  lax.fori_loop(0, CH, zfill, 0)

  def mkidx(row0, clamp):
    def bidx(g, _):
      v = row0 + g * 16 + lanes
      if clamp:
        v = jnp.minimum(v, SUM_ACC_R - 1)
      idx_v[pl.ds(g * 16, 16)] = v
      return 0

    lax.fori_loop(0, CH // 16, bidx, 0)

  def zinit(k, _):
    mkidx(s * zrows + k * CH, True)
    pltpu.sync_copy(rows32_v, acc_sh.at[idx_v])
    return 0

  lax.fori_loop(0, (zrows + CH - 1) // CH, zinit, 0)
  plsc.subcore_barrier()

  def chunk(i, _):
    off = s * EPT + i * CH
    pltpu.sync_copy(src_hbm.at[pl.ds(off, CH)], src_v)
    pltpu.sync_copy(dst_hbm.at[pl.ds(off, CH)], dst_v)

    def vb(g, _):
      d = dst_v[pl.ds(g * 16, 16)]
      dp = d - base
      owned = plsc.bitcast(dp, jnp.uint32) < jnp.uint32(HALF)
      idx_v[pl.ds(g * 16, 16)] = jnp.where(owned, dp, HALF + lanes)
      return 0

    lax.fori_loop(0, CH // 16, vb, 0)
    pltpu.async_copy(xs_hbm.at[src_v], rows_v, sem).wait()

    def repack(e, _):
      rows32_v[e, pl.ds(0, 16)] = rows_v[e, pl.ds(0, 16)]
      rows32_v[e, pl.ds(16, 16)] = rows_v[e, pl.ds(16, 16)]
      return 0

    lax.fori_loop(0, CH, repack, 0)
    pltpu.sync_copy(rows32_v, acc_sh.at[idx_v], add=True)
    return 0

  lax.fori_loop(0, EPT // CH, chunk, 0)
  plsc.subcore_barrier()

  def wb(k, _):
    mkidx(s * zrows + k * CH, False)
    pltpu.async_copy(acc_sh.at[idx_v], rows32_v, sem).wait()
    pltpu.sync_copy(rows32_v, out_hbm.at[c, pl.ds(s * zrows + k * CH, CH)])
    return 0

  lax.fori_loop(0, zrows // CH, wb, 0)
  toff = s * zrows + (zrows // CH) * CH
  idx8_v[...] = jnp.minimum(toff + lanes, SUM_ACC_R - 1)
  pltpu.async_copy(acc_sh.at[idx8_v], rows8_v, sem).wait()
  pltpu.sync_copy(rows8_v.at[pl.ds(0, zrows % CH)],
                  out_hbm.at[c, pl.ds(toff, zrows % CH)])


@functools.partial(
    pl.kernel,
    out_type=(jax.ShapeDtypeStruct((NC, SUM_ACC_R, 16), jnp.float32),
              jax.ShapeDtypeStruct((NC, SUM_ACC_R, 16), jnp.float32)),
    mesh=_mesh(),
    scratch_types=[
        pltpu.VMEM_SHARED((SUM_ACC_R, 16), jnp.float32),
        pltpu.VMEM_SHARED((SUM_ACC_R, 16), jnp.float32),
        pltpu.VMEM((CH,), jnp.int32),
        pltpu.VMEM((CH,), jnp.int32),
        pltpu.VMEM((CH, 16), jnp.float32),
        pltpu.VMEM((CH, 16), jnp.float32),
        pltpu.VMEM((16,), jnp.int32),
        pltpu.VMEM((16, 16), jnp.float32),
        pltpu.SemaphoreType.DMA,
    ],
    compiler_params=pltpu.CompilerParams(needs_layout_passes=False),
)
def _sc_counts(dst1_hbm, dst3_hbm, out1_hbm, out3_hbm,
               acc1_sh, acc3_sh, dst_v, idx_v, ones_v, zb_v, idx8_v,
               rows8_v, sem):
  c = lax.axis_index("c")
  s = lax.axis_index("s")
  base = c * HALF
  zrows = SUM_ACC_R // NS
  lanes = lax.iota(jnp.int32, 16)

  one = jnp.full((16,), 1.0, jnp.float32)
  zv = jnp.zeros((16,), jnp.float32)

  def fill(i, _):
    ones_v[i, :] = one
    zb_v[i, :] = zv
    return 0

  lax.fori_loop(0, CH, fill, 0)

  def mkidx(row0, clamp):
    def bidx(g, _):
      v = row0 + g * 16 + lanes
      if clamp:
        v = jnp.minimum(v, SUM_ACC_R - 1)
      idx_v[pl.ds(g * 16, 16)] = v
      return 0

    lax.fori_loop(0, CH // 16, bidx, 0)

  for acc_sh in (acc1_sh, acc3_sh):

    def zinit(k, _):
      mkidx(s * zrows + k * CH, True)
      pltpu.sync_copy(zb_v, acc_sh.at[idx_v])
      return 0

    lax.fori_loop(0, (zrows + CH - 1) // CH, zinit, 0)

  plsc.subcore_barrier()

  for dst_hbm, acc_sh in ((dst1_hbm, acc1_sh), (dst3_hbm, acc3_sh)):

    def chunk(i, _):
      off = s * EPT + i * CH
      pltpu.sync_copy(dst_hbm.at[pl.ds(off, CH)], dst_v)

      def vb(g, _):
        d = dst_v[pl.ds(g * 16, 16)]
        dp = d - base
        owned = plsc.bitcast(dp, jnp.uint32) < jnp.uint32(HALF)
        idx_v[pl.ds(g * 16, 16)] = jnp.where(owned, dp, HALF + lanes)
        return 0

      lax.fori_loop(0, CH // 16, vb, 0)
      pltpu.sync_copy(ones_v, acc_sh.at[idx_v], add=True)
      return 0

    lax.fori_loop(0, EPT // CH, chunk, 0)

  plsc.subcore_barrier()

  for acc_sh, o_hbm in ((acc1_sh, out1_hbm), (acc3_sh, out3_hbm)):

    def wb(k, _):
      mkidx(s * zrows + k * CH, False)
      pltpu.async_copy(acc_sh.at[idx_v], zb_v, sem).wait()
      pltpu.sync_copy(zb_v, o_hbm.at[c, pl.ds(s * zrows + k * CH, CH)])
      return 0

    lax.fori_loop(0, zrows // CH, wb, 0)
    toff = s * zrows + (zrows // CH) * CH
    idx8_v[...] = jnp.minimum(toff + lanes, SUM_ACC_R - 1)
    pltpu.async_copy(acc_sh.at[idx8_v], rows8_v, sem).wait()
    pltpu.sync_copy(rows8_v.at[pl.ds(0, zrows % CH)],
                    o_hbm.at[c, pl.ds(toff, zrows % CH)])


@functools.partial(
    pl.kernel,
    out_type=jax.ShapeDtypeStruct((NC * NS * MOUT,), jnp.float32),
    mesh=_mesh(),
    scratch_types=[
        pltpu.VMEM((MAFL,), jnp.float32),
        pltpu.VMEM((MCH,), jnp.int32),
        pltpu.VMEM((MCH,), jnp.int32),
        pltpu.VMEM((MCAP,), jnp.int32),
        pltpu.VMEM((MCAP,), jnp.int32),
        pltpu.VMEM((MP, 128), jnp.float32),
        pltpu.SemaphoreType.DMA,
    ],
    compiler_params=pltpu.CompilerParams(needs_layout_passes=False),
)
def _sc_segmax(y_hbm, src_hbm, dst_hbm, ninf_hbm, out_hbm,
               acc_v, src_v, dst_v, csrc_v, cdst_v, rows_v, sem):
  c = lax.axis_index("c")
  s = lax.axis_index("s")
  wid = s * NC + c
  base = wid * BLK

  pltpu.sync_copy(ninf_hbm, acc_v)

  lanes = lax.iota(jnp.int32, 16)

  def initc(g, _):
    cdst_v[pl.ds(g * 16, 16)] = BLK + lanes
    csrc_v[pl.ds(g * 16, 16)] = lanes * 512 + g
    return 0

  lax.fori_loop(0, MCAP // 16, initc, 0)

  def fire():
    pltpu.async_copy(y_hbm.at[csrc_v.at[pl.ds(0, MP)]], rows_v, sem).wait()

    def rmw(e, _):
      dp = cdst_v[pl.ds(e, 16)][0]
      a0 = acc_v[pl.ds(dp * D, 16)]
      r0 = rows_v[e, pl.ds(0, 16)]
      acc_v[pl.ds(dp * D, 16)] = jnp.maximum(a0, r0)
      a1 = acc_v[pl.ds(dp * D + 16, 16)]
      r1 = rows_v[e, pl.ds(16, 16)]
      acc_v[pl.ds(dp * D + 16, 16)] = jnp.maximum(a1, r1)
      return 0

    lax.fori_loop(0, MP, rmw, 0)

    def shift(g, _):
      cdst_v[pl.ds(g * 16, 16)] = cdst_v[pl.ds(MP + g * 16, 16)]
      csrc_v[pl.ds(g * 16, 16)] = csrc_v[pl.ds(MP + g * 16, 16)]
      return 0

    lax.fori_loop(0, (MCAP - MP) // 16, shift, 0)

  def fire_n(n):
    lax.fori_loop(0, n, lambda i, _: (fire(), 0)[1], 0)

  def chunk(i, cur):
    off = i * MCH
    pltpu.sync_copy(src_hbm.at[pl.ds(off, MCH)], src_v)
    pltpu.sync_copy(dst_hbm.at[pl.ds(off, MCH)], dst_v)

    def vb(g, cur):
      d = dst_v[pl.ds(g * 16, 16)]
      sv = src_v[pl.ds(g * 16, 16)]
      dp = d - base
      owned = plsc.bitcast(dp, jnp.uint32) < jnp.uint32(BLK)
      cs = plsc.cumsum(jnp.where(owned, jnp.int32(1), jnp.int32(0)))
      pos = jnp.where(owned, cur + cs - 1, jnp.int32(MCAP - 16) + lanes)
      plsc.store_scatter(cdst_v, [pos], jnp.where(owned, dp, jnp.int32(BLK)))
      plsc.store_scatter(csrc_v, [pos], jnp.where(owned, sv, lanes * 97 + 8))
      return cur + cs[15]

    cur = lax.fori_loop(0, MCH // 16, vb, cur)
    nf = cur // MP
    fire_n(nf)
    return cur - nf * MP

  cur = lax.fori_loop(0, E // MCH, chunk, jnp.int32(0))
  fire()

  pltpu.sync_copy(acc_v.at[pl.ds(0, MOUT)], out_hbm.at[pl.ds(wid * MOUT, MOUT)])


def _prep_body(x_ref, sid_ref, w4_ref, w5_ref, xs_ref, xsp_ref, y4_ref,
               y5_ref):
  xs = x_ref[...] * (1.0 + sid_ref[...].astype(jnp.float32))
  xs_ref[...] = xs
  pad = jnp.zeros((xs.shape[0], 128 - D), jnp.float32)
  xsp_ref[...] = jnp.concatenate([xs, pad], axis=1)
  y4 = jnp.dot(xs, w4_ref[...], preferred_element_type=jnp.float32)
  y5 = jnp.dot(xs, w5_ref[...], preferred_element_type=jnp.float32)
  y4_ref[...] = jnp.concatenate([y4, pad], axis=1)
  y5_ref[...] = jnp.concatenate([y5, pad], axis=1)


def _prep(x, sid, w4, w5):
  blk = _ROWBLK
  return pl.pallas_call(
      _prep_body,
      grid=(_GRID,),
      in_specs=[
          pl.BlockSpec((blk, D), lambda i: (i, 0)),
          pl.BlockSpec((blk, 1), lambda i: (i, 0)),
          pl.BlockSpec((D, D), lambda i: (0, 0)),
          pl.BlockSpec((D, D), lambda i: (0, 0)),
      ],
      out_specs=[
          pl.BlockSpec((blk, D), lambda i: (i, 0)),
          pl.BlockSpec((blk, 128), lambda i: (i, 0)),
          pl.BlockSpec((blk, 128), lambda i: (i, 0)),
          pl.BlockSpec((blk, 128), lambda i: (i, 0)),
      ],
      out_shape=[jax.ShapeDtypeStruct((N, D), jnp.float32),
                 jax.ShapeDtypeStruct((N, 128), jnp.float32),
                 jax.ShapeDtypeStruct((N, 128), jnp.float32),
                 jax.ShapeDtypeStruct((N, 128), jnp.float32)],
  )(x, sid, w4, w5)


def _finish_body(xs_ref, s1_ref, s2_ref, s3_ref, c1_ref, c3_ref, m4_ref,
                 m5_ref, w1_ref, w2_ref, w3_ref, lhw_ref, lhb_ref, dw1_ref,
                 db1_ref, dw2_ref, db2_ref, out_ref):
  f32 = jnp.float32
  c1 = jnp.clip(c1_ref[...][:, 0:1], 1.0, None)
  c3 = jnp.clip(c3_ref[...][:, 0:1], 1.0, None)
  m4 = m4_ref[...]
  m4 = jnp.where(jnp.isfinite(m4), m4, 0.0)
  m5 = m5_ref[...]
  m5 = jnp.where(jnp.isfinite(m5), m5, 0.0)
  h = (xs_ref[...]
       + jnp.dot(s1_ref[...], w1_ref[...], preferred_element_type=f32) / c1
       + jnp.dot(s2_ref[...], w2_ref[...], preferred_element_type=f32)
       + jnp.dot(s3_ref[...], w3_ref[...], preferred_element_type=f32) / c3
       + m4 + m5)
  h = jnp.maximum(h, 0.0)
  feat = jnp.dot(h, lhw_ref[...], preferred_element_type=f32) + lhb_ref[...]
  z = jnp.maximum(
      jnp.dot(feat, dw1_ref[...], preferred_element_type=f32) + db1_ref[...],
      0.0)
  logit = jnp.dot(z, dw2_ref[...], preferred_element_type=f32) + db2_ref[...]
  out_ref[...] = jax.nn.sigmoid(logit)


def _finish(xs, s1, s2, s3, c1, c3, m4, m5, w1, w2, w3, lhw, lhb, dw1, db1,
            dw2, db2):
  blk = _ROWBLK
  row = lambda r, cdim: pl.BlockSpec((blk, cdim), lambda i: (i, 0))
  full = lambda a: pl.BlockSpec(a.shape, lambda i: (0,) * a.ndim)
  return pl.pallas_call(
      _finish_body,
      grid=(_GRID,),
      in_specs=[
          row(xs, D), row(s1, D), row(s2, D), row(s3, D),
          row(c1, 16), row(c3, 16), row(m4, D), row(m5, D),
          full(w1), full(w2), full(w3), full(lhw), full(lhb),
          full(dw1), full(db1), full(dw2), full(db2),
      ],
      out_specs=pl.BlockSpec((blk, 1), lambda i: (i, 0)),
      out_shape=jax.ShapeDtypeStruct((N, 1), jnp.float32),
  )(xs, s1, s2, s3, c1, c3, m4, m5, w1, w2, w3, lhw, lhb, dw1, db1, dw2, db2)


def kernel(x_stroke, edge_intersects, edge_temp_previous, edge_represented_by,
           edge_brepcoplanar, edge_strokecoplanar, sketch_strokes_id,
           W_intersects, W_temp_previous, W_represented_by, W_brepcoplanar,
           W_strokecoplanar, local_head_w, local_head_b, dec_w1, dec_b1,
           dec_w2, dec_b2):
  xs, xsp, y4, y5 = _prep(x_stroke, sketch_strokes_id, W_brepcoplanar,
                          W_strokecoplanar)

  ninf = jnp.full((MAFL,), -jnp.inf, jnp.float32)

  s1 = _sc_segsum(xsp, edge_intersects[0], edge_intersects[1])
  s2 = _sc_segsum(xsp, edge_temp_previous[0], edge_temp_previous[1])
  s3 = _sc_segsum(xsp, edge_represented_by[0], edge_represented_by[1])
  c1, c3 = _sc_counts(edge_intersects[1], edge_represented_by[1])
  m4 = _sc_segmax(y4, edge_brepcoplanar[0], edge_brepcoplanar[1], ninf)
  m5 = _sc_segmax(y5, edge_strokecoplanar[0], edge_strokecoplanar[1], ninf)

  s1, s2, s3 = (t[:, :HALF].reshape(N, D) for t in (s1, s2, s3))
  c1, c3 = (t[:, :HALF].reshape(N, 16) for t in (c1, c3))
  m4, m5 = (t.reshape(NC * NS, BLK + 3, D)[:, :BLK].reshape(N, D)
            for t in (m4, m5))

  return _finish(xs, s1, s2, s3, c1, c3, m4, m5, W_intersects,
                 W_temp_previous, W_represented_by, local_head_w,
                 local_head_b.reshape(1, -1), dec_w1, dec_b1.reshape(1, -1),
                 dec_w2, dec_b2.reshape(1, -1))
